# baseline (speedup 1.0000x reference)
import functools

import jax
import jax.numpy as jnp
from jax import lax
from jax.experimental import pallas as pl
from jax.experimental.pallas import tpu as pltpu
from jax.experimental.pallas import tpu_sc as plsc

N = 2048
D = 1024
E = 8
I = 4096
LANES = 128
M = 256
NBLK = 24
PAD = NBLK * M
KCH = 2
ICH = I // KCH
NC, NS, L = 2, 16, 16
NW = NC * NS
TPW = N // NW
CH = 32


def _router_body(x_ref, gw_ref, gb_ref,
                 pos0_ref, pos1_ref, wa_ref, wb_ref, gid_ref, valid_ref,
                 fetch_ref, slot_ref):
    xv = x_ref[...]
    logits8 = jnp.dot(xv, gw_ref[...], preferred_element_type=jnp.float32)
    logits8 = logits8 + gb_ref[...]
    lane_pad = lax.broadcasted_iota(jnp.int32, (N, LANES), 1)
    logits = jnp.where(lane_pad < E,
                       jnp.pad(logits8, ((0, 0), (0, LANES - E))), -1e30)
    mx = jnp.max(logits, axis=1, keepdims=True)
    ex = jnp.exp(logits - mx)
    probs = ex / jnp.sum(ex, axis=1, keepdims=True)

    lane = lax.broadcasted_iota(jnp.int32, (N, LANES), 1)
    m1 = jnp.max(probs, axis=1, keepdims=True)
    i1 = jnp.min(jnp.where(probs == m1, lane, LANES), axis=1, keepdims=True)
    probs2 = jnp.where(lane == i1, -1.0, probs)
    m2 = jnp.max(probs2, axis=1, keepdims=True)
    i2 = jnp.min(jnp.where(probs2 == m2, lane, LANES), axis=1, keepdims=True)
    ssum = m1 + m2 + 1e-6
    ones_r = jnp.ones((1, 16), jnp.float32)
    wa_ref[...] = (m1 / ssum) * ones_r
    wb_ref[...] = (m2 / ssum) * ones_r

    oh0 = (lane == i1).astype(jnp.float32)
    oh1 = (lane == i2).astype(jnp.float32)

    ri = lax.broadcasted_iota(jnp.int32, (LANES, LANES), 0)
    ci = lax.broadcasted_iota(jnp.int32, (LANES, LANES), 1)
    tril = (ci < ri).astype(jnp.float32)
    triu = (ri < ci).astype(jnp.float32)

    nch = N // LANES
    carry0 = jnp.zeros((1, LANES), jnp.float32)
    carry1 = jnp.zeros((1, LANES), jnp.float32)
    r0_chunks = []
    r1_chunks = []
    for c in range(nch):
        blk0 = oh0[c * LANES:(c + 1) * LANES]
        blk1 = oh1[c * LANES:(c + 1) * LANES]
        r0_chunks.append(jnp.dot(tril, blk0, preferred_element_type=jnp.float32) + carry0)
        r1_chunks.append(jnp.dot(tril, blk1, preferred_element_type=jnp.float32) + carry1)
        carry0 = carry0 + jnp.sum(blk0, axis=0, keepdims=True)
        carry1 = carry1 + jnp.sum(blk1, axis=0, keepdims=True)
    rank0 = jnp.concatenate(r0_chunks, axis=0)
    rank1 = jnp.concatenate(r1_chunks, axis=0)
    cnt0 = carry0
    cnt = carry0 + carry1

    pcnt = jnp.ceil(cnt / M) * M
    off = jnp.dot(pcnt, triu, preferred_element_type=jnp.float32)

    pos0 = jnp.sum(oh0 * (off + rank0), axis=1, keepdims=True)
    pos1 = jnp.sum(oh1 * (off + cnt0 + rank1), axis=1, keepdims=True)
    pos0_ref[...] = pos0.astype(jnp.int32)
    pos1_ref[...] = pos1.astype(jnp.int32)

    bs = off / M
    brow = lax.broadcasted_iota(jnp.int32, (32, LANES), 0).astype(jnp.float32)
    lane32 = lax.broadcasted_iota(jnp.int32, (32, LANES), 1)
    started = jnp.where((bs <= brow) & (lane32 < E), 1.0, 0.0)
    gid = jnp.clip(jnp.sum(started, axis=1, keepdims=True) - 1.0, 0.0, E - 1.0)
    sel = (lane32.astype(jnp.float32) == gid).astype(jnp.float32)
    off_e = jnp.sum(sel * off, axis=1, keepdims=True)
    cnt_e = jnp.sum(sel * cnt, axis=1, keepdims=True)
    bcol = lax.broadcasted_iota(jnp.int32, (32, 1), 0).astype(jnp.float32)
    valid = jnp.clip(off_e + cnt_e - M * bcol, 0.0, float(M))
    gid_ref[...] = gid.astype(jnp.int32)
    valid_ref[...] = valid.astype(jnp.int32)

    act = valid > 0.0
    prevgid = jnp.concatenate([jnp.full((1, 1), -1.0, jnp.float32), gid[:-1]], axis=0)
    chg = gid != prevgid
    fetch = jnp.where(chg & act, 1.0, 0.0)
    ri32 = lax.broadcasted_iota(jnp.int32, (32, 32), 0)
    ci32 = lax.broadcasted_iota(jnp.int32, (32, 32), 1)
    trilI = (ci32 <= ri32).astype(jnp.float32)
    runcum = jnp.dot(trilI, fetch, preferred_element_type=jnp.float32)
    r1 = runcum - 1.0
    slot = r1 - 3.0 * jnp.floor(r1 / 3.0)
    fetch_ref[...] = fetch.astype(jnp.int32)
    slot_ref[...] = slot.astype(jnp.int32)


def _router_call(x2, gw_pad, gb_pad):
    return pl.pallas_call(
        _router_body,
        out_shape=[
            jax.ShapeDtypeStruct((N, 1), jnp.int32),
            jax.ShapeDtypeStruct((N, 1), jnp.int32),
            jax.ShapeDtypeStruct((N, 16), jnp.float32),
            jax.ShapeDtypeStruct((N, 16), jnp.float32),
            jax.ShapeDtypeStruct((32, 1), jnp.int32),
            jax.ShapeDtypeStruct((32, 1), jnp.int32),
            jax.ShapeDtypeStruct((32, 1), jnp.int32),
            jax.ShapeDtypeStruct((32, 1), jnp.int32),
        ],
    )(x2, gw_pad, gb_pad)


def _dispatch_body(x_hbm, pos0_hbm, pos1_hbm, xs_hbm,
                   xbuf, pidx, sem0, sem1):
    wid = lax.axis_index("s") * NC + lax.axis_index("c")
    base = wid * TPW
    pltpu.sync_copy(x_hbm.at[pl.ds(base, TPW)], xbuf)
    pltpu.sync_copy(pos0_hbm.at[pl.ds(base, TPW)], pidx.at[0])
    pltpu.sync_copy(pos1_hbm.at[pl.ds(base, TPW)], pidx.at[1])
    c0 = pltpu.async_copy(xbuf, xs_hbm.at[pidx.at[0]], sem0)
    c1 = pltpu.async_copy(xbuf, xs_hbm.at[pidx.at[1]], sem1)
    c0.wait()
    c1.wait()


def _dispatch_call(x2, pos0f, pos1f):
    mesh = plsc.VectorSubcoreMesh(core_axis_name="c", subcore_axis_name="s")
    fn = functools.partial(
        pl.kernel,
        mesh=mesh,
        out_type=jax.ShapeDtypeStruct((PAD, D), jnp.float32),
        scratch_types=[
            pltpu.VMEM((TPW, D), jnp.float32),
            pltpu.VMEM((2, TPW), jnp.int32),
            pltpu.SemaphoreType.DMA,
            pltpu.SemaphoreType.DMA,
        ],
    )(_dispatch_body)
    return fn(x2, pos0f, pos1f)


NDMA = 8
WCH = D // NDMA


def _wdma_start(w_hbm, wbuf, sems, e, slot):
    r = w_hbm.shape[1] // NDMA
    for c in range(NDMA):
        pltpu.make_async_copy(
            w_hbm.at[e, pl.ds(c * r, r)],
            wbuf.at[slot, pl.ds(c * r, r)],
            sems.at[slot, c],
        ).start()


def _wdma_wait(w_hbm, wbuf, sems, e, slot):
    r = w_hbm.shape[1] // NDMA
    for c in range(NDMA):
        pltpu.make_async_copy(
            w_hbm.at[e, pl.ds(c * r, r)],
            wbuf.at[slot, pl.ds(c * r, r)],
            sems.at[slot, c],
        ).wait()


def _next_fetch(smeta, b):
    def cond(i):
        return jnp.logical_and(i < NBLK, smeta[2, jnp.minimum(i, NBLK - 1)] == 0)
    return lax.while_loop(cond, lambda i: i + 1, b + 1)


def _ffn1_body(smeta, xs_ref, w1_hbm, b1_ref, out_ref, wbuf, sems):
    b = pl.program_id(0)
    nvalid = smeta[1, b]
    slot = smeta[3, b]

    @pl.when((b == 0) & (smeta[2, 0] == 1))
    def _():
        _wdma_start(w1_hbm, wbuf, sems, smeta[0, 0], smeta[3, 0])
        nx1 = _next_fetch(smeta, 0)
        nx1c = jnp.minimum(nx1, NBLK - 1)

        @pl.when(nx1 < NBLK)
        def _():
            _wdma_start(w1_hbm, wbuf, sems, smeta[0, nx1c], smeta[3, nx1c])

    @pl.when(smeta[2, b] == 1)
    def _():
        _wdma_wait(w1_hbm, wbuf, sems, smeta[0, b], slot)
        nx = _next_fetch(smeta, b)
        nxc = jnp.minimum(nx, NBLK - 1)
        nx2 = _next_fetch(smeta, nxc)
        nx2c = jnp.minimum(nx2, NBLK - 1)

        @pl.when((nx < NBLK) & (nx2 < NBLK))
        def _():
            _wdma_start(w1_hbm, wbuf, sems, smeta[0, nx2c], smeta[3, nx2c])

    @pl.when(nvalid > 0)
    def _():
        h = jnp.dot(xs_ref[...], wbuf[slot], preferred_element_type=jnp.float32)
        h = jax.nn.gelu(h + b1_ref[0], approximate=True)
        out_ref[...] = h.astype(jnp.bfloat16)


def _ffn2_body(smeta, h_ref, w2_hbm, b2_ref, out_ref, wbuf, sems):
    b = pl.program_id(0)
    nvalid = smeta[1, b]
    slot = smeta[3, b]

    @pl.when((b == 0) & (smeta[2, 0] == 1))
    def _():
        _wdma_start(w2_hbm, wbuf, sems, smeta[0, 0], smeta[3, 0])
        nx1 = _next_fetch(smeta, 0)
        nx1c = jnp.minimum(nx1, NBLK - 1)

        @pl.when(nx1 < NBLK)
        def _():
            _wdma_start(w2_hbm, wbuf, sems, smeta[0, nx1c], smeta[3, nx1c])

    @pl.when(smeta[2, b] == 1)
    def _():
        _wdma_wait(w2_hbm, wbuf, sems, smeta[0, b], slot)
        nx = _next_fetch(smeta, b)
        nxc = jnp.minimum(nx, NBLK - 1)
        nx2 = _next_fetch(smeta, nxc)
        nx2c = jnp.minimum(nx2, NBLK - 1)

        @pl.when((nx < NBLK) & (nx2 < NBLK))
        def _():
            _wdma_start(w2_hbm, wbuf, sems, smeta[0, nx2c], smeta[3, nx2c])

    @pl.when(nvalid > 0)
    def _():
        h32 = h_ref[...].astype(jnp.float32)
        part = jnp.dot(h32, wbuf[slot], preferred_element_type=jnp.float32)
        out_ref[...] = part + b2_ref[0]


def _ffn_call(smeta, xs, w1, b1, w2, b2):
    gs1 = pltpu.PrefetchScalarGridSpec(
        num_scalar_prefetch=1,
        grid=(NBLK,),
        in_specs=[
            pl.BlockSpec((M, D), lambda b, s: (b, 0)),
            pl.BlockSpec(memory_space=pl.ANY),
            pl.BlockSpec((1, 1, I), lambda b, s: (s[0, b], 0, 0)),
        ],
        out_specs=pl.BlockSpec((M, I), lambda b, s: (b, 0)),
        scratch_shapes=[
            pltpu.VMEM((3, D, I), jnp.float32),
            pltpu.SemaphoreType.DMA((3, NDMA)),
        ],
    )
    h = pl.pallas_call(
        _ffn1_body,
        grid_spec=gs1,
        out_shape=jax.ShapeDtypeStruct((PAD, I), jnp.bfloat16),
    )(smeta, xs, w1, b1.reshape(E, 1, I))
    gs2 = pltpu.PrefetchScalarGridSpec(
        num_scalar_prefetch=1,
        grid=(NBLK,),
        in_specs=[
            pl.BlockSpec((M, I), lambda b, s: (b, 0)),
            pl.BlockSpec(memory_space=pl.ANY),
            pl.BlockSpec((1, 1, D), lambda b, s: (s[0, b], 0, 0)),
        ],
        out_specs=pl.BlockSpec((M, D), lambda b, s: (b, 0)),
        scratch_shapes=[
            pltpu.VMEM((3, I, D), jnp.float32),
            pltpu.SemaphoreType.DMA((3, NDMA)),
        ],
    )
    return pl.pallas_call(
        _ffn2_body,
        grid_spec=gs2,
        out_shape=jax.ShapeDtypeStruct((PAD, D), jnp.float32),
    )(smeta, h, w2, b2.reshape(E, 1, D))


def _combine_body(ys_hbm, pos0_hbm, pos1_hbm, wa_hbm, wb_hbm, out_hbm,
                  abuf, bbuf, wabuf, wbbuf, pidx, sem0, sem1):
    wid = lax.axis_index("s") * NC + lax.axis_index("c")
    base = wid * TPW
    for chunk in range(TPW // CH):
        cb = base + chunk * CH
        pltpu.sync_copy(pos0_hbm.at[pl.ds(cb, CH)], pidx.at[0])
        pltpu.sync_copy(pos1_hbm.at[pl.ds(cb, CH)], pidx.at[1])
        pltpu.sync_copy(wa_hbm.at[pl.ds(cb, CH)], wabuf)
        pltpu.sync_copy(wb_hbm.at[pl.ds(cb, CH)], wbbuf)
        ca = pltpu.async_copy(ys_hbm.at[pidx.at[0]], abuf, sem0)
        cbc = pltpu.async_copy(ys_hbm.at[pidx.at[1]], bbuf, sem1)
        ca.wait()
        cbc.wait()
        for r in range(CH):
            av = wabuf[r]
            bv = wbbuf[r]
            def add_body(j, carry):
                for u in range(4):
                    sl = pl.ds((j * 4 + u) * L, L)
                    abuf[r, sl] = abuf[r, sl] * av + bbuf[r, sl] * bv
                return carry
            lax.fori_loop(0, D // (L * 4), add_body, 0)
        pltpu.sync_copy(abuf, out_hbm.at[pl.ds(cb, CH)])


def _combine_call(ys, pos0f, pos1f, wab, wbb):
    mesh = plsc.VectorSubcoreMesh(core_axis_name="c", subcore_axis_name="s")
    fn = functools.partial(
        pl.kernel,
        mesh=mesh,
        out_type=jax.ShapeDtypeStruct((N, D), jnp.float32),
        scratch_types=[
            pltpu.VMEM((CH, D), jnp.float32),
            pltpu.VMEM((CH, D), jnp.float32),
            pltpu.VMEM((CH, 16), jnp.float32),
            pltpu.VMEM((CH, 16), jnp.float32),
            pltpu.VMEM((2, CH), jnp.int32),
            pltpu.SemaphoreType.DMA,
            pltpu.SemaphoreType.DMA,
        ],
    )(_combine_body)
    return fn(ys, pos0f, pos1f, wab, wbb)


def kernel(x, gate_w, gate_b, w1, b1, w2, b2):
    batch, seq, dim = x.shape
    x2 = x.reshape(N, D)
    gw_pad = gate_w
    gb_pad = gate_b.reshape(1, E)

    pos0, pos1, wa, wb, gid, valid, fetch, slot = _router_call(x2, gw_pad, gb_pad)
    pos0f = pos0.reshape(N)
    pos1f = pos1.reshape(N)
    smeta = jnp.concatenate([gid.reshape(1, 32), valid.reshape(1, 32),
                             fetch.reshape(1, 32), slot.reshape(1, 32)], axis=0)

    xs = _dispatch_call(x2, pos0f, pos1f)
    ys = _ffn_call(smeta, xs, w1, b1, w2, b2)
    out = _combine_call(ys, pos0f, pos1f, wa, wb)
    return out.reshape(batch, seq, dim)


# speedup vs baseline: 1.0689x; 1.0689x over previous
import functools

import jax
import jax.numpy as jnp
from jax import lax
from jax.experimental import pallas as pl
from jax.experimental.pallas import tpu as pltpu
from jax.experimental.pallas import tpu_sc as plsc

N = 2048
D = 1024
E = 8
I = 4096
LANES = 128
M = 256
NBLK = 24
PAD = NBLK * M
KCH = 2
ICH = I // KCH
NC, NS, L = 2, 16, 16
NW = NC * NS
TPW = N // NW
CH = 32


def _router_body(x_ref, gw_ref, gb_ref,
                 pos0_ref, pos1_ref, wa_ref, wb_ref, gid_ref, valid_ref,
                 fetch_ref, slot_ref):
    xv = x_ref[...]
    logits8 = jnp.dot(xv, gw_ref[...], preferred_element_type=jnp.float32)
    logits8 = logits8 + gb_ref[...]
    lane_pad = lax.broadcasted_iota(jnp.int32, (N, LANES), 1)
    logits = jnp.where(lane_pad < E,
                       jnp.pad(logits8, ((0, 0), (0, LANES - E))), -1e30)
    mx = jnp.max(logits, axis=1, keepdims=True)
    ex = jnp.exp(logits - mx)
    probs = ex / jnp.sum(ex, axis=1, keepdims=True)

    lane = lax.broadcasted_iota(jnp.int32, (N, LANES), 1)
    m1 = jnp.max(probs, axis=1, keepdims=True)
    i1 = jnp.min(jnp.where(probs == m1, lane, LANES), axis=1, keepdims=True)
    probs2 = jnp.where(lane == i1, -1.0, probs)
    m2 = jnp.max(probs2, axis=1, keepdims=True)
    i2 = jnp.min(jnp.where(probs2 == m2, lane, LANES), axis=1, keepdims=True)
    ssum = m1 + m2 + 1e-6
    wa_ref[...] = m1 / ssum
    wb_ref[...] = m2 / ssum

    oh0 = (lane == i1).astype(jnp.float32)
    oh1 = (lane == i2).astype(jnp.float32)

    ri = lax.broadcasted_iota(jnp.int32, (LANES, LANES), 0)
    ci = lax.broadcasted_iota(jnp.int32, (LANES, LANES), 1)
    tril = (ci < ri).astype(jnp.float32)
    triu = (ri < ci).astype(jnp.float32)

    nch = N // LANES
    carry0 = jnp.zeros((1, LANES), jnp.float32)
    carry1 = jnp.zeros((1, LANES), jnp.float32)
    r0_chunks = []
    r1_chunks = []
    for c in range(nch):
        blk0 = oh0[c * LANES:(c + 1) * LANES]
        blk1 = oh1[c * LANES:(c + 1) * LANES]
        r0_chunks.append(jnp.dot(tril, blk0, preferred_element_type=jnp.float32) + carry0)
        r1_chunks.append(jnp.dot(tril, blk1, preferred_element_type=jnp.float32) + carry1)
        carry0 = carry0 + jnp.sum(blk0, axis=0, keepdims=True)
        carry1 = carry1 + jnp.sum(blk1, axis=0, keepdims=True)
    rank0 = jnp.concatenate(r0_chunks, axis=0)
    rank1 = jnp.concatenate(r1_chunks, axis=0)
    cnt0 = carry0
    cnt = carry0 + carry1

    pcnt = jnp.ceil(cnt / M) * M
    off = jnp.dot(pcnt, triu, preferred_element_type=jnp.float32)

    pos0 = jnp.sum(oh0 * (off + rank0), axis=1, keepdims=True)
    pos1 = jnp.sum(oh1 * (off + cnt0 + rank1), axis=1, keepdims=True)
    pos0_ref[...] = pos0.astype(jnp.int32)
    pos1_ref[...] = pos1.astype(jnp.int32)

    bs = off / M
    brow = lax.broadcasted_iota(jnp.int32, (32, LANES), 0).astype(jnp.float32)
    lane32 = lax.broadcasted_iota(jnp.int32, (32, LANES), 1)
    started = jnp.where((bs <= brow) & (lane32 < E), 1.0, 0.0)
    gid = jnp.clip(jnp.sum(started, axis=1, keepdims=True) - 1.0, 0.0, E - 1.0)
    sel = (lane32.astype(jnp.float32) == gid).astype(jnp.float32)
    off_e = jnp.sum(sel * off, axis=1, keepdims=True)
    cnt_e = jnp.sum(sel * cnt, axis=1, keepdims=True)
    bcol = lax.broadcasted_iota(jnp.int32, (32, 1), 0).astype(jnp.float32)
    valid = jnp.clip(off_e + cnt_e - M * bcol, 0.0, float(M))
    gid_ref[...] = gid.astype(jnp.int32)
    valid_ref[...] = valid.astype(jnp.int32)

    act = valid > 0.0
    prevgid = jnp.concatenate([jnp.full((1, 1), -1.0, jnp.float32), gid[:-1]], axis=0)
    chg = gid != prevgid
    fetch = jnp.where(chg & act, 1.0, 0.0)
    ri32 = lax.broadcasted_iota(jnp.int32, (32, 32), 0)
    ci32 = lax.broadcasted_iota(jnp.int32, (32, 32), 1)
    trilI = (ci32 <= ri32).astype(jnp.float32)
    runcum = jnp.dot(trilI, fetch, preferred_element_type=jnp.float32)
    r1 = runcum - 1.0
    slot = r1 - 2.0 * jnp.floor(r1 / 2.0)
    fetch_ref[...] = fetch.astype(jnp.int32)
    slot_ref[...] = slot.astype(jnp.int32)


def _router_call(x2, gw_pad, gb_pad):
    return pl.pallas_call(
        _router_body,
        out_shape=[
            jax.ShapeDtypeStruct((N, 1), jnp.int32),
            jax.ShapeDtypeStruct((N, 1), jnp.int32),
            jax.ShapeDtypeStruct((N, 1), jnp.float32),
            jax.ShapeDtypeStruct((N, 1), jnp.float32),
            jax.ShapeDtypeStruct((32, 1), jnp.int32),
            jax.ShapeDtypeStruct((32, 1), jnp.int32),
            jax.ShapeDtypeStruct((32, 1), jnp.int32),
            jax.ShapeDtypeStruct((32, 1), jnp.int32),
        ],
    )(x2, gw_pad, gb_pad)


def _dispatch_body(x_hbm, pos0_hbm, pos1_hbm, xs_hbm,
                   xbuf, pidx, sem0, sem1):
    wid = lax.axis_index("s") * NC + lax.axis_index("c")
    base = wid * TPW
    pltpu.sync_copy(x_hbm.at[pl.ds(base, TPW)], xbuf)
    pltpu.sync_copy(pos0_hbm.at[pl.ds(base, TPW)], pidx.at[0])
    pltpu.sync_copy(pos1_hbm.at[pl.ds(base, TPW)], pidx.at[1])
    c0 = pltpu.async_copy(xbuf, xs_hbm.at[pidx.at[0]], sem0)
    c1 = pltpu.async_copy(xbuf, xs_hbm.at[pidx.at[1]], sem1)
    c0.wait()
    c1.wait()


def _dispatch_call(x2, pos0f, pos1f):
    mesh = plsc.VectorSubcoreMesh(core_axis_name="c", subcore_axis_name="s")
    fn = functools.partial(
        pl.kernel,
        mesh=mesh,
        out_type=jax.ShapeDtypeStruct((PAD, D), jnp.float32),
        scratch_types=[
            pltpu.VMEM((TPW, D), jnp.float32),
            pltpu.VMEM((2, TPW), jnp.int32),
            pltpu.SemaphoreType.DMA,
            pltpu.SemaphoreType.DMA,
        ],
    )(_dispatch_body)
    return fn(x2, pos0f, pos1f)


NDMA = 8
WCH = D // NDMA


def _wdma_start(w_hbm, wbuf, sems, e, slot):
    r = w_hbm.shape[1] // NDMA
    for c in range(NDMA):
        pltpu.make_async_copy(
            w_hbm.at[e, pl.ds(c * r, r)],
            wbuf.at[slot, pl.ds(c * r, r)],
            sems.at[slot, c],
        ).start()


def _wdma_wait(w_hbm, wbuf, sems, e, slot):
    r = w_hbm.shape[1] // NDMA
    for c in range(NDMA):
        pltpu.make_async_copy(
            w_hbm.at[e, pl.ds(c * r, r)],
            wbuf.at[slot, pl.ds(c * r, r)],
            sems.at[slot, c],
        ).wait()


def _next_fetch(smeta, b):
    def cond(i):
        return jnp.logical_and(i < NBLK, smeta[2, jnp.minimum(i, NBLK - 1)] == 0)
    return lax.while_loop(cond, lambda i: i + 1, b + 1)


def _ffn1_body(smeta, xs_ref, w1_hbm, b1_ref, out_ref, wbuf, sems):
    b = pl.program_id(0)
    nvalid = smeta[1, b]
    slot = smeta[3, b]

    @pl.when((b == 0) & (smeta[2, 0] == 1))
    def _():
        _wdma_start(w1_hbm, wbuf, sems, smeta[0, 0], smeta[3, 0])

    @pl.when(smeta[2, b] == 1)
    def _():
        _wdma_wait(w1_hbm, wbuf, sems, smeta[0, b], slot)
        nx = _next_fetch(smeta, b)
        nxc = jnp.minimum(nx, NBLK - 1)

        @pl.when(nx < NBLK)
        def _():
            _wdma_start(w1_hbm, wbuf, sems, smeta[0, nxc], smeta[3, nxc])

    @pl.when(nvalid > 0)
    def _():
        h = jnp.dot(xs_ref[...], wbuf[slot], preferred_element_type=jnp.float32)
        h = jax.nn.gelu(h + b1_ref[0], approximate=True)
        out_ref[...] = h.astype(jnp.bfloat16)


def _ffn2_body(smeta, h_ref, w2_hbm, b2_ref, out_ref, wbuf, sems):
    b = pl.program_id(0)
    nvalid = smeta[1, b]
    slot = smeta[3, b]

    @pl.when((b == 0) & (smeta[2, 0] == 1))
    def _():
        _wdma_start(w2_hbm, wbuf, sems, smeta[0, 0], smeta[3, 0])

    @pl.when(smeta[2, b] == 1)
    def _():
        _wdma_wait(w2_hbm, wbuf, sems, smeta[0, b], slot)
        nx = _next_fetch(smeta, b)
        nxc = jnp.minimum(nx, NBLK - 1)

        @pl.when(nx < NBLK)
        def _():
            _wdma_start(w2_hbm, wbuf, sems, smeta[0, nxc], smeta[3, nxc])

    @pl.when(nvalid > 0)
    def _():
        part = jnp.dot(h_ref[...], wbuf[slot], preferred_element_type=jnp.float32)
        out_ref[...] = part + b2_ref[0]


def _ffn_call(smeta, xs, w1, b1, w2, b2):
    gs1 = pltpu.PrefetchScalarGridSpec(
        num_scalar_prefetch=1,
        grid=(NBLK,),
        in_specs=[
            pl.BlockSpec((M, D), lambda b, s: (b, 0)),
            pl.BlockSpec(memory_space=pl.ANY),
            pl.BlockSpec((1, 1, I), lambda b, s: (s[0, b], 0, 0)),
        ],
        out_specs=pl.BlockSpec((M, I), lambda b, s: (b, 0)),
        scratch_shapes=[
            pltpu.VMEM((2, D, I), jnp.float32),
            pltpu.SemaphoreType.DMA((2, NDMA)),
        ],
    )
    h = pl.pallas_call(
        _ffn1_body,
        grid_spec=gs1,
        out_shape=jax.ShapeDtypeStruct((PAD, I), jnp.bfloat16),
    )(smeta, xs, w1, b1.reshape(E, 1, I))
    gs2 = pltpu.PrefetchScalarGridSpec(
        num_scalar_prefetch=1,
        grid=(NBLK,),
        in_specs=[
            pl.BlockSpec((M, I), lambda b, s: (b, 0)),
            pl.BlockSpec(memory_space=pl.ANY),
            pl.BlockSpec((1, 1, D), lambda b, s: (s[0, b], 0, 0)),
        ],
        out_specs=pl.BlockSpec((M, D), lambda b, s: (b, 0)),
        scratch_shapes=[
            pltpu.VMEM((2, I, D), jnp.float32),
            pltpu.SemaphoreType.DMA((2, NDMA)),
        ],
    )
    return pl.pallas_call(
        _ffn2_body,
        grid_spec=gs2,
        out_shape=jax.ShapeDtypeStruct((PAD, D), jnp.float32),
    )(smeta, h, w2, b2.reshape(E, 1, D))


def _combine_body(ys_hbm, pos0_hbm, pos1_hbm, ga_hbm, gb_hbm,
                  abuf, bbuf, pidx, sem0, sem1):
    wid = lax.axis_index("s") * NC + lax.axis_index("c")
    base = wid * TPW
    for chunk in range(TPW // CH):
        cb = base + chunk * CH
        pltpu.sync_copy(pos0_hbm.at[pl.ds(cb, CH)], pidx.at[0])
        pltpu.sync_copy(pos1_hbm.at[pl.ds(cb, CH)], pidx.at[1])
        ca = pltpu.async_copy(ys_hbm.at[pidx.at[0]], abuf, sem0)
        cbc = pltpu.async_copy(ys_hbm.at[pidx.at[1]], bbuf, sem1)
        ca.wait()
        cbc.wait()
        pltpu.sync_copy(abuf, ga_hbm.at[pl.ds(cb, CH)])
        pltpu.sync_copy(bbuf, gb_hbm.at[pl.ds(cb, CH)])


def _combine_call(ys, pos0f, pos1f):
    mesh = plsc.VectorSubcoreMesh(core_axis_name="c", subcore_axis_name="s")
    fn = functools.partial(
        pl.kernel,
        mesh=mesh,
        out_type=[
            jax.ShapeDtypeStruct((N, D), jnp.float32),
            jax.ShapeDtypeStruct((N, D), jnp.float32),
        ],
        scratch_types=[
            pltpu.VMEM((CH, D), jnp.float32),
            pltpu.VMEM((CH, D), jnp.float32),
            pltpu.VMEM((2, CH), jnp.int32),
            pltpu.SemaphoreType.DMA,
            pltpu.SemaphoreType.DMA,
        ],
    )(_combine_body)
    return fn(ys, pos0f, pos1f)


def _epilogue_body(ga_ref, gb_ref, wa_ref, wb_ref, out_ref):
    out_ref[...] = ga_ref[...] * wa_ref[...] + gb_ref[...] * wb_ref[...]


def _epilogue_call(ga, gb, wa, wb):
    return pl.pallas_call(
        _epilogue_body,
        out_shape=jax.ShapeDtypeStruct((N, D), jnp.float32),
    )(ga, gb, wa, wb)


def kernel(x, gate_w, gate_b, w1, b1, w2, b2):
    batch, seq, dim = x.shape
    x2 = x.reshape(N, D)
    gw_pad = gate_w
    gb_pad = gate_b.reshape(1, E)

    pos0, pos1, wa, wb, gid, valid, fetch, slot = _router_call(x2, gw_pad, gb_pad)
    pos0f = pos0.reshape(N)
    pos1f = pos1.reshape(N)
    smeta = jnp.concatenate([gid.reshape(1, 32), valid.reshape(1, 32),
                             fetch.reshape(1, 32), slot.reshape(1, 32)], axis=0)

    xs = _dispatch_call(x2, pos0f, pos1f)
    ys = _ffn_call(smeta, xs, w1, b1, w2, b2)
    ga, gb = _combine_call(ys, pos0f, pos1f)
    out = _epilogue_call(ga, gb, wa, wb)
    return out.reshape(batch, seq, dim)


# speedup vs baseline: 1.0770x; 1.0075x over previous
import functools

import jax
import jax.numpy as jnp
from jax import lax
from jax.experimental import pallas as pl
from jax.experimental.pallas import tpu as pltpu
from jax.experimental.pallas import tpu_sc as plsc

N = 2048
D = 1024
E = 8
I = 4096
LANES = 128
M = 256
NBLK = 24
PAD = NBLK * M
KCH = 2
ICH = I // KCH
NC, NS, L = 2, 16, 16
NW = NC * NS
TPW = N // NW
CH = 32


def _router_body(x_ref, gw_ref, gb_ref,
                 pos0_ref, pos1_ref, wa_ref, wb_ref, gid_ref, valid_ref,
                 fetch_ref, slot_ref):
    xv = x_ref[...]
    logits8 = jnp.dot(xv, gw_ref[...], preferred_element_type=jnp.float32)
    logits8 = logits8 + gb_ref[...]
    lane_pad = lax.broadcasted_iota(jnp.int32, (N, LANES), 1)
    logits = jnp.where(lane_pad < E,
                       jnp.pad(logits8, ((0, 0), (0, LANES - E))), -1e30)
    mx = jnp.max(logits, axis=1, keepdims=True)
    ex = jnp.exp(logits - mx)
    probs = ex / jnp.sum(ex, axis=1, keepdims=True)

    lane = lax.broadcasted_iota(jnp.int32, (N, LANES), 1)
    m1 = jnp.max(probs, axis=1, keepdims=True)
    i1 = jnp.min(jnp.where(probs == m1, lane, LANES), axis=1, keepdims=True)
    probs2 = jnp.where(lane == i1, -1.0, probs)
    m2 = jnp.max(probs2, axis=1, keepdims=True)
    i2 = jnp.min(jnp.where(probs2 == m2, lane, LANES), axis=1, keepdims=True)
    ssum = m1 + m2 + 1e-6
    wa_ref[...] = m1 / ssum
    wb_ref[...] = m2 / ssum

    oh0 = (lane == i1).astype(jnp.float32)
    oh1 = (lane == i2).astype(jnp.float32)

    ri = lax.broadcasted_iota(jnp.int32, (LANES, LANES), 0)
    ci = lax.broadcasted_iota(jnp.int32, (LANES, LANES), 1)
    tril = (ci < ri).astype(jnp.float32)
    triu = (ri < ci).astype(jnp.float32)

    nch = N // LANES
    carry0 = jnp.zeros((1, LANES), jnp.float32)
    carry1 = jnp.zeros((1, LANES), jnp.float32)
    r0_chunks = []
    r1_chunks = []
    for c in range(nch):
        blk0 = oh0[c * LANES:(c + 1) * LANES]
        blk1 = oh1[c * LANES:(c + 1) * LANES]
        r0_chunks.append(jnp.dot(tril, blk0, preferred_element_type=jnp.float32) + carry0)
        r1_chunks.append(jnp.dot(tril, blk1, preferred_element_type=jnp.float32) + carry1)
        carry0 = carry0 + jnp.sum(blk0, axis=0, keepdims=True)
        carry1 = carry1 + jnp.sum(blk1, axis=0, keepdims=True)
    rank0 = jnp.concatenate(r0_chunks, axis=0)
    rank1 = jnp.concatenate(r1_chunks, axis=0)
    cnt0 = carry0
    cnt = carry0 + carry1

    pcnt = jnp.ceil(cnt / M) * M
    off = jnp.dot(pcnt, triu, preferred_element_type=jnp.float32)

    pos0 = jnp.sum(oh0 * (off + rank0), axis=1, keepdims=True)
    pos1 = jnp.sum(oh1 * (off + cnt0 + rank1), axis=1, keepdims=True)
    pos0_ref[...] = pos0.astype(jnp.int32)
    pos1_ref[...] = pos1.astype(jnp.int32)

    bs = off / M
    brow = lax.broadcasted_iota(jnp.int32, (32, LANES), 0).astype(jnp.float32)
    lane32 = lax.broadcasted_iota(jnp.int32, (32, LANES), 1)
    started = jnp.where((bs <= brow) & (lane32 < E), 1.0, 0.0)
    gid = jnp.clip(jnp.sum(started, axis=1, keepdims=True) - 1.0, 0.0, E - 1.0)
    sel = (lane32.astype(jnp.float32) == gid).astype(jnp.float32)
    off_e = jnp.sum(sel * off, axis=1, keepdims=True)
    cnt_e = jnp.sum(sel * cnt, axis=1, keepdims=True)
    bcol = lax.broadcasted_iota(jnp.int32, (32, 1), 0).astype(jnp.float32)
    valid = jnp.clip(off_e + cnt_e - M * bcol, 0.0, float(M))
    gid_ref[...] = gid.astype(jnp.int32)
    valid_ref[...] = valid.astype(jnp.int32)

    act = valid > 0.0
    prevgid = jnp.concatenate([jnp.full((1, 1), -1.0, jnp.float32), gid[:-1]], axis=0)
    chg = gid != prevgid
    fetch = jnp.where(chg & act, 1.0, 0.0)
    ri32 = lax.broadcasted_iota(jnp.int32, (32, 32), 0)
    ci32 = lax.broadcasted_iota(jnp.int32, (32, 32), 1)
    trilI = (ci32 <= ri32).astype(jnp.float32)
    runcum = jnp.dot(trilI, fetch, preferred_element_type=jnp.float32)
    r1 = runcum - 1.0
    slot = r1 - 2.0 * jnp.floor(r1 / 2.0)
    fetch_ref[...] = fetch.astype(jnp.int32)
    slot_ref[...] = slot.astype(jnp.int32)


def _router_call(x2, gw_pad, gb_pad):
    return pl.pallas_call(
        _router_body,
        out_shape=[
            jax.ShapeDtypeStruct((N, 1), jnp.int32),
            jax.ShapeDtypeStruct((N, 1), jnp.int32),
            jax.ShapeDtypeStruct((N, 1), jnp.float32),
            jax.ShapeDtypeStruct((N, 1), jnp.float32),
            jax.ShapeDtypeStruct((32, 1), jnp.int32),
            jax.ShapeDtypeStruct((32, 1), jnp.int32),
            jax.ShapeDtypeStruct((32, 1), jnp.int32),
            jax.ShapeDtypeStruct((32, 1), jnp.int32),
        ],
    )(x2, gw_pad, gb_pad)


def _dispatch_body(x_hbm, pos0_hbm, pos1_hbm, xs_hbm,
                   xbuf, pidx, sem0, sem1):
    wid = lax.axis_index("s") * NC + lax.axis_index("c")
    base = wid * TPW
    pltpu.sync_copy(x_hbm.at[pl.ds(base, TPW)], xbuf)
    pltpu.sync_copy(pos0_hbm.at[pl.ds(base, TPW)], pidx.at[0])
    pltpu.sync_copy(pos1_hbm.at[pl.ds(base, TPW)], pidx.at[1])
    c0 = pltpu.async_copy(xbuf, xs_hbm.at[pidx.at[0]], sem0)
    c1 = pltpu.async_copy(xbuf, xs_hbm.at[pidx.at[1]], sem1)
    c0.wait()
    c1.wait()


def _dispatch_call(x2, pos0f, pos1f):
    mesh = plsc.VectorSubcoreMesh(core_axis_name="c", subcore_axis_name="s")
    fn = functools.partial(
        pl.kernel,
        mesh=mesh,
        out_type=jax.ShapeDtypeStruct((PAD, D), jnp.float32),
        scratch_types=[
            pltpu.VMEM((TPW, D), jnp.float32),
            pltpu.VMEM((2, TPW), jnp.int32),
            pltpu.SemaphoreType.DMA,
            pltpu.SemaphoreType.DMA,
        ],
    )(_dispatch_body)
    return fn(x2, pos0f, pos1f)


NDMA = 8
WCH = D // NDMA


def _wdma_start(w_hbm, wbuf, sems, e, slot):
    r = w_hbm.shape[1] // NDMA
    for c in range(NDMA):
        pltpu.make_async_copy(
            w_hbm.at[e, pl.ds(c * r, r)],
            wbuf.at[slot, pl.ds(c * r, r)],
            sems.at[slot, c],
        ).start()


def _wdma_wait(w_hbm, wbuf, sems, e, slot):
    r = w_hbm.shape[1] // NDMA
    for c in range(NDMA):
        pltpu.make_async_copy(
            w_hbm.at[e, pl.ds(c * r, r)],
            wbuf.at[slot, pl.ds(c * r, r)],
            sems.at[slot, c],
        ).wait()


def _next_fetch(smeta, b):
    def cond(i):
        return jnp.logical_and(i < NBLK, smeta[2, jnp.minimum(i, NBLK - 1)] == 0)
    return lax.while_loop(cond, lambda i: i + 1, b + 1)


def _ffn1_body(smeta, xs_ref, w1_hbm, b1_ref, out_ref, wbuf, sems):
    b = pl.program_id(0)
    nvalid = smeta[1, b]
    slot = smeta[3, b]

    @pl.when((b == 0) & (smeta[2, 0] == 1))
    def _():
        _wdma_start(w1_hbm, wbuf, sems, smeta[0, 0], smeta[3, 0])

    @pl.when(smeta[2, b] == 1)
    def _():
        nx = _next_fetch(smeta, b)
        nxc = jnp.minimum(nx, NBLK - 1)

        @pl.when(nx < NBLK)
        def _():
            _wdma_start(w1_hbm, wbuf, sems, smeta[0, nxc], smeta[3, nxc])

        _wdma_wait(w1_hbm, wbuf, sems, smeta[0, b], slot)

    @pl.when(nvalid > 0)
    def _():
        h = jnp.dot(xs_ref[...], wbuf[slot], preferred_element_type=jnp.float32)
        h = jax.nn.gelu(h + b1_ref[0], approximate=True)
        out_ref[...] = h.astype(jnp.bfloat16)


def _ffn2_body(smeta, h_ref, w2_hbm, b2_ref, out_ref, wbuf, sems):
    b = pl.program_id(0)
    nvalid = smeta[1, b]
    slot = smeta[3, b]

    @pl.when((b == 0) & (smeta[2, 0] == 1))
    def _():
        _wdma_start(w2_hbm, wbuf, sems, smeta[0, 0], smeta[3, 0])

    @pl.when(smeta[2, b] == 1)
    def _():
        nx = _next_fetch(smeta, b)
        nxc = jnp.minimum(nx, NBLK - 1)

        @pl.when(nx < NBLK)
        def _():
            _wdma_start(w2_hbm, wbuf, sems, smeta[0, nxc], smeta[3, nxc])

        _wdma_wait(w2_hbm, wbuf, sems, smeta[0, b], slot)

    @pl.when(nvalid > 0)
    def _():
        part = jnp.dot(h_ref[...], wbuf[slot], preferred_element_type=jnp.float32)
        out_ref[...] = part + b2_ref[0]


def _ffn_call(smeta, xs, w1, b1, w2, b2):
    gs1 = pltpu.PrefetchScalarGridSpec(
        num_scalar_prefetch=1,
        grid=(NBLK,),
        in_specs=[
            pl.BlockSpec((M, D), lambda b, s: (b, 0)),
            pl.BlockSpec(memory_space=pl.ANY),
            pl.BlockSpec((1, 1, I), lambda b, s: (s[0, b], 0, 0)),
        ],
        out_specs=pl.BlockSpec((M, I), lambda b, s: (b, 0)),
        scratch_shapes=[
            pltpu.VMEM((2, D, I), jnp.float32),
            pltpu.SemaphoreType.DMA((2, NDMA)),
        ],
    )
    h = pl.pallas_call(
        _ffn1_body,
        grid_spec=gs1,
        out_shape=jax.ShapeDtypeStruct((PAD, I), jnp.bfloat16),
    )(smeta, xs, w1, b1.reshape(E, 1, I))
    gs2 = pltpu.PrefetchScalarGridSpec(
        num_scalar_prefetch=1,
        grid=(NBLK,),
        in_specs=[
            pl.BlockSpec((M, I), lambda b, s: (b, 0)),
            pl.BlockSpec(memory_space=pl.ANY),
            pl.BlockSpec((1, 1, D), lambda b, s: (s[0, b], 0, 0)),
        ],
        out_specs=pl.BlockSpec((M, D), lambda b, s: (b, 0)),
        scratch_shapes=[
            pltpu.VMEM((2, I, D), jnp.float32),
            pltpu.SemaphoreType.DMA((2, NDMA)),
        ],
    )
    return pl.pallas_call(
        _ffn2_body,
        grid_spec=gs2,
        out_shape=jax.ShapeDtypeStruct((PAD, D), jnp.float32),
    )(smeta, h, w2, b2.reshape(E, 1, D))


def _combine_body(ys_hbm, pos0_hbm, pos1_hbm, ga_hbm, gb_hbm,
                  abuf, bbuf, pidx, sem0, sem1):
    wid = lax.axis_index("s") * NC + lax.axis_index("c")
    base = wid * TPW
    for chunk in range(TPW // CH):
        cb = base + chunk * CH
        pltpu.sync_copy(pos0_hbm.at[pl.ds(cb, CH)], pidx.at[0])
        pltpu.sync_copy(pos1_hbm.at[pl.ds(cb, CH)], pidx.at[1])
        ca = pltpu.async_copy(ys_hbm.at[pidx.at[0]], abuf, sem0)
        cbc = pltpu.async_copy(ys_hbm.at[pidx.at[1]], bbuf, sem1)
        ca.wait()
        cbc.wait()
        pltpu.sync_copy(abuf, ga_hbm.at[pl.ds(cb, CH)])
        pltpu.sync_copy(bbuf, gb_hbm.at[pl.ds(cb, CH)])


def _combine_call(ys, pos0f, pos1f):
    mesh = plsc.VectorSubcoreMesh(core_axis_name="c", subcore_axis_name="s")
    fn = functools.partial(
        pl.kernel,
        mesh=mesh,
        out_type=[
            jax.ShapeDtypeStruct((N, D), jnp.float32),
            jax.ShapeDtypeStruct((N, D), jnp.float32),
        ],
        scratch_types=[
            pltpu.VMEM((CH, D), jnp.float32),
            pltpu.VMEM((CH, D), jnp.float32),
            pltpu.VMEM((2, CH), jnp.int32),
            pltpu.SemaphoreType.DMA,
            pltpu.SemaphoreType.DMA,
        ],
    )(_combine_body)
    return fn(ys, pos0f, pos1f)


def _epilogue_body(ga_ref, gb_ref, wa_ref, wb_ref, out_ref):
    out_ref[...] = ga_ref[...] * wa_ref[...] + gb_ref[...] * wb_ref[...]


def _epilogue_call(ga, gb, wa, wb):
    return pl.pallas_call(
        _epilogue_body,
        out_shape=jax.ShapeDtypeStruct((N, D), jnp.float32),
    )(ga, gb, wa, wb)


def kernel(x, gate_w, gate_b, w1, b1, w2, b2):
    batch, seq, dim = x.shape
    x2 = x.reshape(N, D)
    gw_pad = gate_w
    gb_pad = gate_b.reshape(1, E)

    pos0, pos1, wa, wb, gid, valid, fetch, slot = _router_call(x2, gw_pad, gb_pad)
    pos0f = pos0.reshape(N)
    pos1f = pos1.reshape(N)
    smeta = jnp.concatenate([gid.reshape(1, 32), valid.reshape(1, 32),
                             fetch.reshape(1, 32), slot.reshape(1, 32)], axis=0)

    xs = _dispatch_call(x2, pos0f, pos1f)
    ys = _ffn_call(smeta, xs, w1, b1, w2, b2)
    ga, gb = _combine_call(ys, pos0f, pos1f)
    out = _epilogue_call(ga, gb, wa, wb)
    return out.reshape(batch, seq, dim)
